# baseline (speedup 1.0000x reference)
import functools

import jax
import jax.numpy as jnp
from jax.experimental import pallas as pl
from jax.experimental.pallas import tpu as pltpu

MICRO_BATCHES = 2
B, S, D, E = 2, 2048, 1024, 8
H = 4 * D
KH = 4
HT = H // KH
TOK = (B * S) // (MICRO_BATCHES * E)


def _gelu_exact(v):
    return 0.5 * v * (1.0 + jax.lax.erf(v * 0.7071067811865476))


def _block_kernel(x_ref, ln_g_ref, ln_b_ref, attn_W_ref, attn_b_ref,
                  fc1_W_ref, fc1_b_ref, fc2_W_ref, fc2_b_ref,
                  next_W_ref, next_b_ref, out_ref,
                  xattn_s, h_s, acc_s):
    j = pl.program_id(2)

    @pl.when(j == 0)
    def _pre():
        xb = x_ref[...]
        mu = jnp.mean(xb, axis=1, keepdims=True)
        var = jnp.mean((xb - mu) ** 2, axis=1, keepdims=True)
        xn = (xb - mu) * jax.lax.rsqrt(var + 1e-5)
        xn = xn * ln_g_ref[...] + ln_b_ref[...]
        xa = jnp.dot(xn.astype(jnp.bfloat16), attn_W_ref[...],
                     preferred_element_type=jnp.float32)
        xa = xa + attn_b_ref[...] + xb
        xattn_s[...] = xa.astype(jnp.bfloat16)

    @pl.when(j > 0)
    def _fc2():
        contrib = jnp.dot(h_s[(j - 1) % 2], fc2_W_ref[0],
                          preferred_element_type=jnp.float32)

        @pl.when(j == 1)
        def _init():
            acc_s[...] = contrib

        @pl.when(j > 1)
        def _accum():
            acc_s[...] = acc_s[...] + contrib

    @pl.when(j < KH)
    def _fc1():
        h = jnp.dot(xattn_s[...], fc1_W_ref[0],
                    preferred_element_type=jnp.float32)
        h_s[j % 2] = _gelu_exact(h + fc1_b_ref[0]).astype(jnp.bfloat16)

    @pl.when(j == KH)
    def _post():
        y = acc_s[...] + fc2_b_ref[0]
        z = jnp.dot(y.astype(jnp.bfloat16), next_W_ref[...],
                    preferred_element_type=jnp.float32)
        out_ref[...] = _gelu_exact(z + next_b_ref[...])


@functools.partial(jax.jit, static_argnames=())
def kernel(x, ln_g, ln_b, attn_W, attn_b, gate_W, fc1_W, fc1_b, fc2_W,
           fc2_b, next_W, next_b):
    del gate_W
    b, s, d = x.shape
    x_flat = x.reshape(b * s, d)

    attn_W_b = attn_W.astype(jnp.bfloat16)
    fc1_W_b = fc1_W.astype(jnp.bfloat16)
    fc2_W_b = fc2_W.astype(jnp.bfloat16)
    next_W_b = next_W.astype(jnp.bfloat16)

    ln_g2 = ln_g.reshape(1, D)
    ln_b2 = ln_b.reshape(1, D)
    attn_b2 = attn_b.reshape(1, D)
    next_b2 = next_b.reshape(1, D)
    fc1_b3 = fc1_b.reshape(E, 1, H)
    fc2_b3 = fc2_b.reshape(E, 1, D)

    grid = (E, MICRO_BATCHES, KH + 1)

    out = pl.pallas_call(
        _block_kernel,
        grid=grid,
        in_specs=[
            pl.BlockSpec((TOK, D), lambda e, c, j: (c * E + e, 0)),
            pl.BlockSpec((1, D), lambda e, c, j: (0, 0)),
            pl.BlockSpec((1, D), lambda e, c, j: (0, 0)),
            pl.BlockSpec((D, D), lambda e, c, j: (0, 0)),
            pl.BlockSpec((1, D), lambda e, c, j: (0, 0)),
            pl.BlockSpec((1, D, HT),
                         lambda e, c, j: (e, 0, jnp.minimum(j, KH - 1))),
            pl.BlockSpec((1, 1, HT),
                         lambda e, c, j: (e, 0, jnp.minimum(j, KH - 1))),
            pl.BlockSpec((1, HT, D),
                         lambda e, c, j: (e, jnp.maximum(j - 1, 0), 0)),
            pl.BlockSpec((1, 1, D), lambda e, c, j: (e, 0, 0)),
            pl.BlockSpec((D, D), lambda e, c, j: (0, 0)),
            pl.BlockSpec((1, D), lambda e, c, j: (0, 0)),
        ],
        out_specs=pl.BlockSpec((TOK, D), lambda e, c, j: (c * E + e, 0)),
        out_shape=jax.ShapeDtypeStruct((b * s, d), jnp.float32),
        scratch_shapes=[
            pltpu.VMEM((TOK, D), jnp.bfloat16),
            pltpu.VMEM((2, TOK, HT), jnp.bfloat16),
            pltpu.VMEM((TOK, D), jnp.float32),
        ],
        compiler_params=pltpu.CompilerParams(
            dimension_semantics=("parallel", "parallel", "arbitrary"),
        ),
    )(x_flat, ln_g2, ln_b2, attn_W_b, attn_b2,
      fc1_W_b, fc1_b3, fc2_W_b, fc2_b3, next_W_b, next_b2)

    return out.reshape(b, s, d)


# speedup vs baseline: 1.2222x; 1.2222x over previous
import functools

import jax
import jax.numpy as jnp
from jax.experimental import pallas as pl
from jax.experimental.pallas import tpu as pltpu

MICRO_BATCHES = 2
B, S, D, E = 2, 2048, 1024, 8
H = 4 * D
KH = 4
HT = H // KH
TOK = (B * S) // (MICRO_BATCHES * E)
M = MICRO_BATCHES * TOK


def _gelu_exact(v):
    return 0.5 * v * (1.0 + jax.lax.erf(v * 0.7071067811865476))


def _block_kernel(x_ref, ln_g_ref, ln_b_ref, attn_W_ref, attn_b_ref,
                  fc1_W_ref, fc1_b_ref, fc2_W_ref, fc2_b_ref,
                  next_W_ref, next_b_ref, out_ref,
                  xattn_s, acc_s):
    j = pl.program_id(1)

    @pl.when(j == 0)
    def _pre():
        xb = x_ref[...].reshape(M, D)
        mu = jnp.mean(xb, axis=1, keepdims=True)
        var = jnp.mean((xb - mu) ** 2, axis=1, keepdims=True)
        xn = (xb - mu) * jax.lax.rsqrt(var + 1e-5)
        xn = xn * ln_g_ref[...] + ln_b_ref[...]
        xa = jnp.dot(xn.astype(jnp.bfloat16), attn_W_ref[...],
                     preferred_element_type=jnp.float32)
        xa = xa + attn_b_ref[...] + xb
        xattn_s[...] = xa.astype(jnp.bfloat16)

    h = jnp.dot(xattn_s[...], fc1_W_ref[0],
                preferred_element_type=jnp.float32)
    h = _gelu_exact(h + fc1_b_ref[0])
    contrib = jnp.dot(h.astype(jnp.bfloat16), fc2_W_ref[0],
                      preferred_element_type=jnp.float32)

    @pl.when(j == 0)
    def _init():
        acc_s[...] = contrib

    @pl.when(j > 0)
    def _accum():
        acc_s[...] = acc_s[...] + contrib

    @pl.when(j == KH - 1)
    def _post():
        y = acc_s[...] + fc2_b_ref[0]
        z = jnp.dot(y.astype(jnp.bfloat16), next_W_ref[...],
                    preferred_element_type=jnp.float32)
        out_ref[...] = _gelu_exact(z + next_b_ref[...]).reshape(
            MICRO_BATCHES, 1, TOK, D)


@functools.partial(jax.jit, static_argnames=())
def kernel(x, ln_g, ln_b, attn_W, attn_b, gate_W, fc1_W, fc1_b, fc2_W,
           fc2_b, next_W, next_b):
    del gate_W
    b, s, d = x.shape
    x4 = x.reshape(MICRO_BATCHES, E, TOK, D)

    attn_W_b = attn_W.astype(jnp.bfloat16)
    fc1_W_b = fc1_W.astype(jnp.bfloat16)
    fc2_W_b = fc2_W.astype(jnp.bfloat16)
    next_W_b = next_W.astype(jnp.bfloat16)

    ln_g2 = ln_g.reshape(1, D)
    ln_b2 = ln_b.reshape(1, D)
    attn_b2 = attn_b.reshape(1, D)
    next_b2 = next_b.reshape(1, D)
    fc1_b3 = fc1_b.reshape(E, 1, H)
    fc2_b3 = fc2_b.reshape(E, 1, D)

    grid = (E, KH)

    out = pl.pallas_call(
        _block_kernel,
        grid=grid,
        in_specs=[
            pl.BlockSpec((MICRO_BATCHES, 1, TOK, D),
                         lambda e, j: (0, e, 0, 0)),
            pl.BlockSpec((1, D), lambda e, j: (0, 0)),
            pl.BlockSpec((1, D), lambda e, j: (0, 0)),
            pl.BlockSpec((D, D), lambda e, j: (0, 0)),
            pl.BlockSpec((1, D), lambda e, j: (0, 0)),
            pl.BlockSpec((1, D, HT), lambda e, j: (e, 0, j)),
            pl.BlockSpec((1, 1, HT), lambda e, j: (e, 0, j)),
            pl.BlockSpec((1, HT, D), lambda e, j: (e, j, 0)),
            pl.BlockSpec((1, 1, D), lambda e, j: (e, 0, 0)),
            pl.BlockSpec((D, D), lambda e, j: (0, 0)),
            pl.BlockSpec((1, D), lambda e, j: (0, 0)),
        ],
        out_specs=pl.BlockSpec((MICRO_BATCHES, 1, TOK, D),
                               lambda e, j: (0, e, 0, 0)),
        out_shape=jax.ShapeDtypeStruct((MICRO_BATCHES, E, TOK, D),
                                       jnp.float32),
        scratch_shapes=[
            pltpu.VMEM((M, D), jnp.bfloat16),
            pltpu.VMEM((M, D), jnp.float32),
        ],
        compiler_params=pltpu.CompilerParams(
            dimension_semantics=("parallel", "arbitrary"),
        ),
    )(x4, ln_g2, ln_b2, attn_W_b, attn_b2,
      fc1_W_b, fc1_b3, fc2_W_b, fc2_b3, next_W_b, next_b2)

    return out.reshape(b, s, d)
